# Initial kernel scaffold; baseline (speedup 1.0000x reference)
import jax
import jax.numpy as jnp
from jax.experimental import pallas as pl


def kernel(ast_type, ast_edge, ast_batch, cfg_type, cfg_edge, cfg_batch, struct_sem, ast_emb, ast_W1, ast_b1, ast_W2, ast_b2, cfg_emb, cfg_W1, cfg_b1, cfg_W2, cfg_b2, sem_W, sem_b, fuse1_W, fuse1_b, fuse2_W, fuse2_b, ln_g, ln_b, cls_W, cls_b):
    raise NotImplementedError("write your pallas kernel here")


# speedup vs baseline: 14.5379x; 14.5379x over previous
import functools

import jax
import jax.numpy as jnp
from jax import lax
from jax.experimental import pallas as pl
from jax.experimental.pallas import tpu as pltpu
from jax.experimental.pallas import tpu_sc as plsc

F32 = jnp.float32
I32 = jnp.int32

N = 50000
E = 800000
G = 256
D = 64
HH = 32
NPAD = 51200
NPT = 3200
EPT = 50176
EPAD = 16 * EPT
ECH = EPAD // 128
CPT = 392
DUMP = NPAD - 1
ACCR = NPAD
ACC2R = 53248
GP = 272
EPS = 1e-5

_PREC = lax.Precision.HIGHEST


def _mesh():
  return plsc.VectorSubcoreMesh(
      core_axis_name="c", subcore_axis_name="s", num_cores=2, num_subcores=16)


def _sc_params():
  return pltpu.CompilerParams(use_tc_tiling_on_sc=False)


def _rsqrt16(x):
  i = lax.bitcast_convert_type(x, I32)
  i = jnp.int32(0x5F3759DF) - lax.shift_right_logical(i, 1)
  y = lax.bitcast_convert_type(i, F32)
  for _ in range(3):
    y = y * (1.5 - 0.5 * x * y * y)
  return y


def _zero_rows(ref, nrows, width):
  z = jnp.zeros((16,), F32)
  nv = width // 16

  @pl.loop(0, nrows)
  def _(r):
    for v in range(nv):
      ref[r, pl.ds(v * 16, 16)] = z


def _edge_pipeline(edges, ibufa, ibufb, semia, semib, base, nchunks, fire_one):
  @pl.loop(0, nchunks // 2)
  def _(i):
    p = base + 2 * i
    q = p + 1

    @pl.when(i == 0)
    def _():
      pltpu.sync_copy(edges.at[p], ibufa)

    @pl.when(i > 0)
    def _():
      pltpu.make_async_copy(edges.at[p], ibufa, semia).wait()

    hb = pltpu.async_copy(edges.at[q], ibufb, semib)
    ha = fire_one(ibufa, 0)
    hb.wait()
    hc = fire_one(ibufb, 1)
    for h in ha:
      h.wait()

    @pl.when(i < nchunks // 2 - 1)
    def _():
      pltpu.async_copy(edges.at[p + 2], ibufa, semia)

    for h in hc:
      h.wait()


@functools.lru_cache(maxsize=None)
def _prep_kernel():
  def body(edges, types, t1l, t1r,
           dinv_out, xwsl, xwsr,
           acc2, zbuf, ones, ibufa, ibufb, dbuf, tbuf, gbuf,
           semia, semib, sems):
    c = lax.axis_index("c")
    s = lax.axis_index("s")

    _zero_rows(zbuf, 256, 16)
    one = jnp.ones((16,), F32)

    @pl.loop(0, 128)
    def _(r):
      ones[r, pl.ds(0, 16)] = one

    @pl.loop(0, 13)
    def _(k):
      pltpu.sync_copy(zbuf, acc2.at[pl.ds((s * 13 + k) * 256, 256)])

    plsc.subcore_barrier()

    def fire_hist(ibuf, which):
      del which
      return [pltpu.async_copy(ones, acc2.at[ibuf.at[1]], sems, add=True)]

    _edge_pipeline(edges, ibufa, ibufb, semia, semib, s * CPT, CPT, fire_hist)

    plsc.subcore_barrier()

    pltpu.sync_copy(acc2.at[pl.ds(s * NPT, NPT)], dbuf)

    @pl.loop(0, NPT)
    def _(r):
      v = dbuf[r, pl.ds(0, 16)]
      dbuf[r, pl.ds(0, 16)] = _rsqrt16(v + 1.0)

    @pl.when(c == 0)
    def _():
      pltpu.sync_copy(dbuf, dinv_out.at[pl.ds(s * NPT, NPT)])

    def xws_phase(t1, out):
      @pl.loop(0, 25)
      def _(ch):
        row = s * 25 + ch
        pltpu.sync_copy(types.at[row], tbuf)
        pltpu.async_copy(t1.at[tbuf], gbuf, sems).wait()

        @pl.loop(0, 128)
        def _(r):
          v = dbuf[ch * 128 + r, pl.ds(0, 16)]
          gbuf[r, pl.ds(0, 16)] = gbuf[r, pl.ds(0, 16)] * v
          gbuf[r, pl.ds(16, 16)] = gbuf[r, pl.ds(16, 16)] * v

        pltpu.sync_copy(gbuf, out.at[pl.ds(s * NPT + ch * 128, 128)])

    @pl.when(c == 0)
    def _():
      xws_phase(t1l, xwsl)

    @pl.when(c == 1)
    def _():
      xws_phase(t1r, xwsr)

  return pl.kernel(
      body,
      out_type=(
          jax.ShapeDtypeStruct((NPAD, 16), F32),
          jax.ShapeDtypeStruct((NPAD, HH), F32),
          jax.ShapeDtypeStruct((NPAD, HH), F32),
      ),
      mesh=_mesh(),
      scratch_types=[
          pltpu.VMEM_SHARED((ACC2R, 16), F32),
          pltpu.VMEM((256, 16), F32),
          pltpu.VMEM((128, 16), F32),
          pltpu.VMEM((2, 128), I32),
          pltpu.VMEM((2, 128), I32),
          pltpu.VMEM((NPT, 16), F32),
          pltpu.VMEM((128,), I32),
          pltpu.VMEM((128, HH), F32),
          pltpu.SemaphoreType.DMA,
          pltpu.SemaphoreType.DMA,
          pltpu.SemaphoreType.DMA,
      ],
      compiler_params=_sc_params(),
      name="cgnl_prep",
  )


@functools.lru_cache(maxsize=None)
def _conv_kernel(relu: bool, pool: bool):
  def body(*refs):
    if pool:
      (edges, xwsl, xwsr, dinv, bias, batch,
       outl, outr,
       acc, ibufa, ibufb, rbufa, rbufb, abuf, wbuf, vbuf, sbuf,
       poolt, bbuf,
       semia, semib, semg, semsc, semw) = refs
    else:
      (edges, xwsl, xwsr, dinv, bias,
       outl, outr,
       acc, ibufa, ibufb, rbufa, rbufb, abuf, wbuf, vbuf, sbuf,
       semia, semib, semg, semsc, semw) = refs

    c = lax.axis_index("c")
    s = lax.axis_index("s")

    pltpu.sync_copy(bias, sbuf)

    _zero_rows(rbufa, 128, HH)
    zbase = s * NPT

    @pl.loop(0, 25)
    def _(k):
      pltpu.sync_copy(rbufa, acc.at[pl.ds(zbase + k * 128, 128)])

    if pool:
      _zero_rows(poolt, GP, HH)

    plsc.subcore_barrier()

    def run_core(xws, out, ci):
      b0 = sbuf[ci, 0, pl.ds(0, 16)]
      b1 = sbuf[ci, 1, pl.ds(0, 16)]

      def fire_conv(ibuf, which):
        rbuf = rbufb if which else rbufa
        ssem = semw if which else semsc
        pltpu.async_copy(xws.at[ibuf.at[0]], rbuf, semg).wait()
        return [pltpu.async_copy(rbuf, acc.at[ibuf.at[1]], ssem, add=True)]

      _edge_pipeline(edges, ibufa, ibufb, semia, semib, s * CPT, CPT,
                     fire_conv)

      plsc.subcore_barrier()

      @pl.loop(0, 50)
      def _(ch):
        nbase = s * NPT + ch * 64
        pltpu.sync_copy(acc.at[pl.ds(nbase, 64)], abuf)
        pltpu.sync_copy(xws.at[pl.ds(nbase, 64)], wbuf)
        pltpu.sync_copy(dinv.at[pl.ds(nbase, 64)], vbuf)
        if pool:
          pltpu.sync_copy(batch.at[s * 50 + ch], bbuf)

        def row_h(r):
          d = vbuf[r, pl.ds(0, 16)]
          v0 = (abuf[r, pl.ds(0, 16)] + wbuf[r, pl.ds(0, 16)]) * d + b0
          v1 = (abuf[r, pl.ds(16, 16)] + wbuf[r, pl.ds(16, 16)]) * d + b1
          if relu:
            v0 = jnp.maximum(v0, 0.0)
            v1 = jnp.maximum(v1, 0.0)
          return v0, v1

        if pool:
          @pl.loop(0, 4)
          def _(rg):
            gvec = bbuf[pl.ds(rg * 16, 16)]
            for lane in range(16):
              r = rg * 16 + lane
              v0, v1 = row_h(r)
              gidx = gvec[lane]
              poolt[gidx, pl.ds(0, 16)] = poolt[gidx, pl.ds(0, 16)] + v0
              poolt[gidx, pl.ds(16, 16)] = poolt[gidx, pl.ds(16, 16)] + v1
        else:
          @pl.loop(0, 64)
          def _(r):
            v0, v1 = row_h(r)
            abuf[r, pl.ds(0, 16)] = v0
            abuf[r, pl.ds(16, 16)] = v1

          pltpu.sync_copy(abuf, out.at[pl.ds(nbase, 64)])

      if pool:
        pltpu.sync_copy(poolt, out.at[s])

    @pl.when(c == 0)
    def _():
      run_core(xwsl, outl, 0)

    @pl.when(c == 1)
    def _():
      run_core(xwsr, outr, 1)

  if pool:
    out_type = (jax.ShapeDtypeStruct((16, GP, HH), F32),
                jax.ShapeDtypeStruct((16, GP, HH), F32))
  else:
    out_type = (jax.ShapeDtypeStruct((NPAD, HH), F32),
                jax.ShapeDtypeStruct((NPAD, HH), F32))
  scratch = [
      pltpu.VMEM_SHARED((ACCR, HH), F32),
      pltpu.VMEM((2, 128), I32),
      pltpu.VMEM((2, 128), I32),
      pltpu.VMEM((128, HH), F32),
      pltpu.VMEM((128, HH), F32),
      pltpu.VMEM((64, HH), F32),
      pltpu.VMEM((64, HH), F32),
      pltpu.VMEM((64, 16), F32),
      pltpu.VMEM((2, 2, 16), F32),
  ]
  if pool:
    scratch += [
        pltpu.VMEM((GP, HH), F32),
        pltpu.VMEM((64,), I32),
    ]
  scratch += [pltpu.SemaphoreType.DMA] * 5

  return pl.kernel(
      body,
      out_type=out_type,
      mesh=_mesh(),
      scratch_types=scratch,
      compiler_params=_sc_params(),
      name=f"cgnl_conv_{int(relu)}{int(pool)}",
  )


def _dg(a, b):
  return lax.dot_general(a, b, (((1,), (1,)), ((), ())),
                         precision=_PREC, preferred_element_type=F32)


@functools.lru_cache(maxsize=None)
def _t1_kernel():
  def body(ea, wa, ec, wc, oa, oc):
    oa[...] = _dg(ea[...], wa[...])
    oc[...] = _dg(ec[...], wc[...])

  return pl.pallas_call(
      body,
      out_shape=(jax.ShapeDtypeStruct((256, D), F32),
                 jax.ShapeDtypeStruct((256, D), F32)),
  )


@functools.lru_cache(maxsize=None)
def _mm_kernel():
  blk = 2048

  def body(hl, hr, w2, dv, ol, orr):
    h = jnp.concatenate([hl[...], hr[...]], axis=1)
    x = _dg(h, w2[...])
    x = x * dv[...][:, 0:1]
    ol[...] = x[:, :HH]
    orr[...] = x[:, HH:]

  nb = NPAD // blk
  return pl.pallas_call(
      body,
      grid=(nb,),
      in_specs=[
          pl.BlockSpec((blk, HH), lambda i: (i, 0)),
          pl.BlockSpec((blk, HH), lambda i: (i, 0)),
          pl.BlockSpec((D, D), lambda i: (0, 0)),
          pl.BlockSpec((blk, 16), lambda i: (i, 0)),
      ],
      out_specs=(pl.BlockSpec((blk, HH), lambda i: (i, 0)),
                 pl.BlockSpec((blk, HH), lambda i: (i, 0))),
      out_shape=(jax.ShapeDtypeStruct((NPAD, HH), F32),
                 jax.ShapeDtypeStruct((NPAD, HH), F32)),
  )


@functools.lru_cache(maxsize=None)
def _head_kernel():
  def body(pal, par, pcl, pcr, ss, semw, semb, f1w, f1b, f2w, f2b,
           lng, lnb, clsw, clsb, out):
    ha = jnp.concatenate([jnp.sum(pal[...], axis=0)[:G],
                          jnp.sum(par[...], axis=0)[:G]], axis=1)
    hc = jnp.concatenate([jnp.sum(pcl[...], axis=0)[:G],
                          jnp.sum(pcr[...], axis=0)[:G]], axis=1)
    f1 = f1w[...]
    z1 = _dg(ha, f1[:, :D]) + _dg(hc, f1[:, D:]) + f1b[...]
    g1 = 1.0 / (1.0 + jnp.exp(-z1))
    hs = g1 * ha + (1.0 - g1) * hc
    hm = jnp.maximum(_dg(ss[...], semw[...]) + semb[...], 0.0)
    f2 = f2w[...]
    z2 = _dg(hs, f2[:, :D]) + _dg(hm, f2[:, D:]) + f2b[...]
    g2 = 1.0 / (1.0 + jnp.exp(-z2))
    h = g2 * hs + (1.0 - g2) * hm
    mu = jnp.mean(h, axis=1, keepdims=True)
    var = jnp.mean((h - mu) ** 2, axis=1, keepdims=True)
    hn = (h - mu) / jnp.sqrt(var + EPS) * lng[...] + lnb[...]
    out[...] = _dg(hn, clsw[...]) + clsb[...]

  return pl.pallas_call(
      body,
      out_shape=jax.ShapeDtypeStruct((G, 2), F32),
  )


def _pack_edges(edge):
  src = jnp.concatenate(
      [edge[0].astype(I32), jnp.full((EPAD - E,), DUMP, I32)])
  dst = jnp.concatenate(
      [edge[1].astype(I32), jnp.full((EPAD - E,), DUMP, I32)])
  return jnp.stack([src.reshape(ECH, 128), dst.reshape(ECH, 128)], axis=1)


def _encoder(edge, types, batch, t1, w2, b1, b2):
  edges = _pack_edges(edge)
  types_r = jnp.pad(types.astype(I32), (0, NPAD - N)).reshape(NPAD // 128, 128)
  batch_r = jnp.pad(batch.astype(I32), (0, NPAD - N),
                    constant_values=G).reshape(NPAD // 64, 64)
  dinv, xw1l, xw1r = _prep_kernel()(edges, types_r, t1[:, :HH], t1[:, HH:])
  h1l, h1r = _conv_kernel(True, False)(
      edges, xw1l, xw1r, dinv, b1.reshape(2, 2, 16))
  xw2l, xw2r = _mm_kernel()(h1l, h1r, w2, dinv)
  pll, plr = _conv_kernel(False, True)(
      edges, xw2l, xw2r, dinv, b2.reshape(2, 2, 16), batch_r)
  return pll, plr


def kernel(ast_type, ast_edge, ast_batch, cfg_type, cfg_edge, cfg_batch,
           struct_sem, ast_emb, ast_W1, ast_b1, ast_W2, ast_b2,
           cfg_emb, cfg_W1, cfg_b1, cfg_W2, cfg_b2,
           sem_W, sem_b, fuse1_W, fuse1_b, fuse2_W, fuse2_b,
           ln_g, ln_b, cls_W, cls_b):
  ea = jnp.pad(ast_emb, ((0, 256 - ast_emb.shape[0]), (0, 0)))
  ec = jnp.pad(cfg_emb, ((0, 256 - cfg_emb.shape[0]), (0, 0)))
  t1a, t1c = _t1_kernel()(ea, ast_W1, ec, cfg_W1)
  pal, par = _encoder(ast_edge, ast_type, ast_batch, t1a, ast_W2,
                      ast_b1, ast_b2)
  pcl, pcr = _encoder(cfg_edge, cfg_type, cfg_batch, t1c, cfg_W2,
                      cfg_b1, cfg_b2)
  return _head_kernel()(
      pal, par, pcl, pcr, struct_sem, sem_W, sem_b.reshape(1, D),
      fuse1_W, fuse1_b.reshape(1, D), fuse2_W, fuse2_b.reshape(1, D),
      ln_g.reshape(1, D), ln_b.reshape(1, D), cls_W, cls_b.reshape(1, 2))


# speedup vs baseline: 24.2035x; 1.6649x over previous
import functools

import jax
import jax.numpy as jnp
from jax import lax
from jax.experimental import pallas as pl
from jax.experimental.pallas import tpu as pltpu
from jax.experimental.pallas import tpu_sc as plsc

F32 = jnp.float32
I32 = jnp.int32

N = 50000
E = 800000
G = 256
D = 64
HH = 32
NPAD = 51200
NPT = 3200
EPT = 50176
EPAD = 16 * EPT
ECH = EPAD // 128
CPT = 392
DUMP = NPAD - 1
ACCR = NPAD
ACC2R = 53248
GP = 272
EPS = 1e-5

_PREC = lax.Precision.HIGHEST


def _mesh():
  return plsc.VectorSubcoreMesh(
      core_axis_name="c", subcore_axis_name="s", num_cores=2, num_subcores=16)


def _sc_params():
  return pltpu.CompilerParams(use_tc_tiling_on_sc=False)


def _rsqrt16(x):
  i = lax.bitcast_convert_type(x, I32)
  i = jnp.int32(0x5F3759DF) - lax.shift_right_logical(i, 1)
  y = lax.bitcast_convert_type(i, F32)
  for _ in range(3):
    y = y * (1.5 - 0.5 * x * y * y)
  return y


def _zero_rows(ref, nrows, width):
  z = jnp.zeros((16,), F32)
  nv = width // 16

  @pl.loop(0, nrows)
  def _(r):
    for v in range(nv):
      ref[r, pl.ds(v * 16, 16)] = z


def _edge_sets(edges, ibufa, ibufb, semia, semib, base, nbuf, niter, do_set):
  @pl.loop(0, niter)
  def _(i):
    cb = base + i * (2 * nbuf)

    @pl.when(i == 0)
    def _():
      pltpu.async_copy(edges.at[pl.ds(cb, nbuf)], ibufa, semia)

    pltpu.make_async_copy(edges.at[pl.ds(cb, nbuf)], ibufa, semia).wait()
    hb = pltpu.async_copy(edges.at[pl.ds(cb + nbuf, nbuf)], ibufb, semib)
    do_set(ibufa)
    hb.wait()

    @pl.when(i < niter - 1)
    def _():
      pltpu.async_copy(edges.at[pl.ds(cb + 2 * nbuf, nbuf)], ibufa, semia)

    do_set(ibufb)


@functools.lru_cache(maxsize=None)
def _prep_kernel():
  def body(edges, types, t1l, t1r,
           dinv_out, xwsl, xwsr,
           acc2, zbuf, ones, ibufa, ibufb, dbuf, tbuf, gbuf,
           semia, semib, sems):
    c = lax.axis_index("c")
    s = lax.axis_index("s")

    _zero_rows(zbuf, 256, 16)
    one = jnp.ones((16,), F32)

    @pl.loop(0, 128)
    def _(r):
      ones[r, pl.ds(0, 16)] = one

    @pl.loop(0, 13)
    def _(k):
      pltpu.sync_copy(zbuf, acc2.at[pl.ds((s * 13 + k) * 256, 256)])

    plsc.subcore_barrier()

    def hist_set(ibuf):
      hs = [pltpu.async_copy(ones, acc2.at[ibuf.at[b].at[1]], sems, add=True)
            for b in range(4)]
      for h in hs:
        h.wait()

    _edge_sets(edges, ibufa, ibufb, semia, semib, s * CPT, 4, CPT // 8,
               hist_set)

    plsc.subcore_barrier()

    pltpu.sync_copy(acc2.at[pl.ds(s * NPT, NPT)], dbuf)

    @pl.loop(0, NPT)
    def _(r):
      v = dbuf[r, pl.ds(0, 16)]
      dbuf[r, pl.ds(0, 16)] = _rsqrt16(v + 1.0)

    @pl.when(c == 0)
    def _():
      pltpu.sync_copy(dbuf, dinv_out.at[pl.ds(s * NPT, NPT)])

    def xws_phase(t1, out):
      @pl.loop(0, 25)
      def _(ch):
        row = s * 25 + ch
        pltpu.sync_copy(types.at[row], tbuf)
        pltpu.async_copy(t1.at[tbuf], gbuf, sems).wait()

        @pl.loop(0, 128)
        def _(r):
          v = dbuf[ch * 128 + r, pl.ds(0, 16)]
          gbuf[r, pl.ds(0, 16)] = gbuf[r, pl.ds(0, 16)] * v
          gbuf[r, pl.ds(16, 16)] = gbuf[r, pl.ds(16, 16)] * v

        pltpu.sync_copy(gbuf, out.at[pl.ds(s * NPT + ch * 128, 128)])

    @pl.when(c == 0)
    def _():
      xws_phase(t1l, xwsl)

    @pl.when(c == 1)
    def _():
      xws_phase(t1r, xwsr)

  return pl.kernel(
      body,
      out_type=(
          jax.ShapeDtypeStruct((NPAD, 16), F32),
          jax.ShapeDtypeStruct((NPAD, HH), F32),
          jax.ShapeDtypeStruct((NPAD, HH), F32),
      ),
      mesh=_mesh(),
      scratch_types=[
          pltpu.VMEM_SHARED((ACC2R, 16), F32),
          pltpu.VMEM((256, 16), F32),
          pltpu.VMEM((128, 16), F32),
          pltpu.VMEM((4, 2, 128), I32),
          pltpu.VMEM((4, 2, 128), I32),
          pltpu.VMEM((NPT, 16), F32),
          pltpu.VMEM((128,), I32),
          pltpu.VMEM((128, HH), F32),
          pltpu.SemaphoreType.DMA,
          pltpu.SemaphoreType.DMA,
          pltpu.SemaphoreType.DMA,
      ],
      compiler_params=_sc_params(),
      name="cgnl_prep",
  )


@functools.lru_cache(maxsize=None)
def _conv_kernel(relu: bool, pool: bool):
  nbuf = 2 if pool else 4
  niter = CPT // (2 * nbuf)

  def body(*refs):
    if pool:
      (edges, xwsl, xwsr, dinv, bias, batch, outl, outr, acc,
       ibufa, ibufb) = refs[:11]
      rbufs = refs[11:11 + nbuf]
      (abuf, vbuf, sbuf, poolt, bbuf,
       semia, semib, semg, semsc) = refs[11 + nbuf:]
    else:
      (edges, xwsl, xwsr, dinv, bias, outl, outr, acc,
       ibufa, ibufb) = refs[:10]
      rbufs = refs[10:10 + nbuf]
      (abuf, vbuf, sbuf,
       semia, semib, semg, semsc) = refs[10 + nbuf:]

    c = lax.axis_index("c")
    s = lax.axis_index("s")

    pltpu.sync_copy(bias, sbuf)
    if pool:
      _zero_rows(poolt, GP, HH)

    def run_core(xws, out, ci):
      b0 = sbuf[ci, 0, pl.ds(0, 16)]
      b1 = sbuf[ci, 1, pl.ds(0, 16)]

      @pl.loop(0, 25)
      def _(k):
        nb = s * NPT + k * 128
        pltpu.sync_copy(xws.at[pl.ds(nb, 128)], acc.at[pl.ds(nb, 128)])

      plsc.subcore_barrier()

      def do_set(ibuf):
        hg = [pltpu.async_copy(xws.at[ibuf.at[b].at[0]], rbufs[b], semg)
              for b in range(nbuf)]
        hs = []
        for b in range(nbuf):
          hg[b].wait()
          hs.append(pltpu.async_copy(rbufs[b], acc.at[ibuf.at[b].at[1]],
                                     semsc, add=True))
        for h in hs:
          h.wait()

      _edge_sets(edges, ibufa, ibufb, semia, semib, s * CPT, nbuf, niter,
                 do_set)

      plsc.subcore_barrier()

      @pl.loop(0, 25)
      def _(ch):
        nbase = s * NPT + ch * 128
        pltpu.sync_copy(acc.at[pl.ds(nbase, 128)], abuf)
        pltpu.sync_copy(dinv.at[pl.ds(nbase, 128)], vbuf)
        if pool:
          pltpu.sync_copy(batch.at[s * 25 + ch], bbuf)

        def row_h(r):
          d = vbuf[r, pl.ds(0, 16)]
          v0 = abuf[r, pl.ds(0, 16)] * d + b0
          v1 = abuf[r, pl.ds(16, 16)] * d + b1
          if relu:
            v0 = jnp.maximum(v0, 0.0)
            v1 = jnp.maximum(v1, 0.0)
          return v0, v1

        if pool:
          @pl.loop(0, 8)
          def _(rg):
            gvec = bbuf[pl.ds(rg * 16, 16)]
            for lane in range(16):
              r = rg * 16 + lane
              v0, v1 = row_h(r)
              gidx = gvec[lane]
              poolt[gidx, pl.ds(0, 16)] = poolt[gidx, pl.ds(0, 16)] + v0
              poolt[gidx, pl.ds(16, 16)] = poolt[gidx, pl.ds(16, 16)] + v1
        else:
          @pl.loop(0, 128)
          def _(r):
            v0, v1 = row_h(r)
            abuf[r, pl.ds(0, 16)] = v0
            abuf[r, pl.ds(16, 16)] = v1

          pltpu.sync_copy(abuf, out.at[pl.ds(nbase, 128)])

      if pool:
        pltpu.sync_copy(poolt, out.at[s])

    @pl.when(c == 0)
    def _():
      run_core(xwsl, outl, 0)

    @pl.when(c == 1)
    def _():
      run_core(xwsr, outr, 1)

  if pool:
    out_type = (jax.ShapeDtypeStruct((16, GP, HH), F32),
                jax.ShapeDtypeStruct((16, GP, HH), F32))
  else:
    out_type = (jax.ShapeDtypeStruct((NPAD, HH), F32),
                jax.ShapeDtypeStruct((NPAD, HH), F32))
  scratch = [
      pltpu.VMEM_SHARED((ACCR, HH), F32),
      pltpu.VMEM((nbuf, 2, 128), I32),
      pltpu.VMEM((nbuf, 2, 128), I32),
  ]
  scratch += [pltpu.VMEM((128, HH), F32) for _ in range(nbuf)]
  scratch += [
      pltpu.VMEM((128, HH), F32),
      pltpu.VMEM((128, 16), F32),
      pltpu.VMEM((2, 2, 16), F32),
  ]
  if pool:
    scratch += [
        pltpu.VMEM((GP, HH), F32),
        pltpu.VMEM((128,), I32),
    ]
  scratch += [pltpu.SemaphoreType.DMA] * 4

  return pl.kernel(
      body,
      out_type=out_type,
      mesh=_mesh(),
      scratch_types=scratch,
      compiler_params=_sc_params(),
      name=f"cgnl_conv_{int(relu)}{int(pool)}",
  )


def _dg(a, b):
  return lax.dot_general(a, b, (((1,), (1,)), ((), ())),
                         precision=_PREC, preferred_element_type=F32)


@functools.lru_cache(maxsize=None)
def _t1_kernel():
  def body(ea, wa, ec, wc, oa, oc):
    oa[...] = _dg(ea[...], wa[...])
    oc[...] = _dg(ec[...], wc[...])

  return pl.pallas_call(
      body,
      out_shape=(jax.ShapeDtypeStruct((256, D), F32),
                 jax.ShapeDtypeStruct((256, D), F32)),
  )


@functools.lru_cache(maxsize=None)
def _mm_kernel():
  blk = 2048

  def body(hl, hr, w2, dv, ol, orr):
    h = jnp.concatenate([hl[...], hr[...]], axis=1)
    x = _dg(h, w2[...])
    x = x * dv[...][:, 0:1]
    ol[...] = x[:, :HH]
    orr[...] = x[:, HH:]

  nb = NPAD // blk
  return pl.pallas_call(
      body,
      grid=(nb,),
      in_specs=[
          pl.BlockSpec((blk, HH), lambda i: (i, 0)),
          pl.BlockSpec((blk, HH), lambda i: (i, 0)),
          pl.BlockSpec((D, D), lambda i: (0, 0)),
          pl.BlockSpec((blk, 16), lambda i: (i, 0)),
      ],
      out_specs=(pl.BlockSpec((blk, HH), lambda i: (i, 0)),
                 pl.BlockSpec((blk, HH), lambda i: (i, 0))),
      out_shape=(jax.ShapeDtypeStruct((NPAD, HH), F32),
                 jax.ShapeDtypeStruct((NPAD, HH), F32)),
  )


@functools.lru_cache(maxsize=None)
def _head_kernel():
  def body(pal, par, pcl, pcr, ss, semw, semb, f1w, f1b, f2w, f2b,
           lng, lnb, clsw, clsb, out):
    ha = jnp.concatenate([jnp.sum(pal[...], axis=0)[:G],
                          jnp.sum(par[...], axis=0)[:G]], axis=1)
    hc = jnp.concatenate([jnp.sum(pcl[...], axis=0)[:G],
                          jnp.sum(pcr[...], axis=0)[:G]], axis=1)
    f1 = f1w[...]
    z1 = _dg(ha, f1[:, :D]) + _dg(hc, f1[:, D:]) + f1b[...]
    g1 = 1.0 / (1.0 + jnp.exp(-z1))
    hs = g1 * ha + (1.0 - g1) * hc
    hm = jnp.maximum(_dg(ss[...], semw[...]) + semb[...], 0.0)
    f2 = f2w[...]
    z2 = _dg(hs, f2[:, :D]) + _dg(hm, f2[:, D:]) + f2b[...]
    g2 = 1.0 / (1.0 + jnp.exp(-z2))
    h = g2 * hs + (1.0 - g2) * hm
    mu = jnp.mean(h, axis=1, keepdims=True)
    var = jnp.mean((h - mu) ** 2, axis=1, keepdims=True)
    hn = (h - mu) / jnp.sqrt(var + EPS) * lng[...] + lnb[...]
    out[...] = _dg(hn, clsw[...]) + clsb[...]

  return pl.pallas_call(
      body,
      out_shape=jax.ShapeDtypeStruct((G, 2), F32),
  )


def _pack_edges(edge):
  src = jnp.concatenate(
      [edge[0].astype(I32), jnp.full((EPAD - E,), DUMP, I32)])
  dst = jnp.concatenate(
      [edge[1].astype(I32), jnp.full((EPAD - E,), DUMP, I32)])
  return jnp.stack([src.reshape(ECH, 128), dst.reshape(ECH, 128)], axis=1)


def _encoder(edge, types, batch, t1, w2, b1, b2):
  edges = _pack_edges(edge)
  types_r = jnp.pad(types.astype(I32), (0, NPAD - N)).reshape(NPAD // 128, 128)
  batch_r = jnp.pad(batch.astype(I32), (0, NPAD - N),
                    constant_values=G).reshape(NPAD // 128, 128)
  dinv, xw1l, xw1r = _prep_kernel()(edges, types_r, t1[:, :HH], t1[:, HH:])
  h1l, h1r = _conv_kernel(True, False)(
      edges, xw1l, xw1r, dinv, b1.reshape(2, 2, 16))
  xw2l, xw2r = _mm_kernel()(h1l, h1r, w2, dinv)
  pll, plr = _conv_kernel(False, True)(
      edges, xw2l, xw2r, dinv, b2.reshape(2, 2, 16), batch_r)
  return pll, plr


def kernel(ast_type, ast_edge, ast_batch, cfg_type, cfg_edge, cfg_batch,
           struct_sem, ast_emb, ast_W1, ast_b1, ast_W2, ast_b2,
           cfg_emb, cfg_W1, cfg_b1, cfg_W2, cfg_b2,
           sem_W, sem_b, fuse1_W, fuse1_b, fuse2_W, fuse2_b,
           ln_g, ln_b, cls_W, cls_b):
  ea = jnp.pad(ast_emb, ((0, 256 - ast_emb.shape[0]), (0, 0)))
  ec = jnp.pad(cfg_emb, ((0, 256 - cfg_emb.shape[0]), (0, 0)))
  t1a, t1c = _t1_kernel()(ea, ast_W1, ec, cfg_W1)
  pal, par = _encoder(ast_edge, ast_type, ast_batch, t1a, ast_W2,
                      ast_b1, ast_b2)
  pcl, pcr = _encoder(cfg_edge, cfg_type, cfg_batch, t1c, cfg_W2,
                      cfg_b1, cfg_b2)
  return _head_kernel()(
      pal, par, pcl, pcr, struct_sem, sem_W, sem_b.reshape(1, D),
      fuse1_W, fuse1_b.reshape(1, D), fuse2_W, fuse2_b.reshape(1, D),
      ln_g.reshape(1, D), ln_b.reshape(1, D), cls_W, cls_b.reshape(1, 2))


# speedup vs baseline: 26.4058x; 1.0910x over previous
import functools

import jax
import jax.numpy as jnp
from jax import lax
from jax.experimental import pallas as pl
from jax.experimental.pallas import tpu as pltpu
from jax.experimental.pallas import tpu_sc as plsc

F32 = jnp.float32
I32 = jnp.int32

N = 50000
E = 800000
G = 256
D = 64
HH = 32
NPAD = 51200
NPT = 3200
EPT = 50176
EPAD = 16 * EPT
ECH = EPAD // 128
CPT = 392
DUMP = NPAD - 1
ACCR = NPAD
ACC2R = 53248
GP = 272
EPS = 1e-5

_PREC = lax.Precision.HIGHEST


def _mesh():
  return plsc.VectorSubcoreMesh(
      core_axis_name="c", subcore_axis_name="s", num_cores=2, num_subcores=16)


def _sc_params():
  return pltpu.CompilerParams(use_tc_tiling_on_sc=False)


def _rsqrt16(x):
  i = lax.bitcast_convert_type(x, I32)
  i = jnp.int32(0x5F3759DF) - lax.shift_right_logical(i, 1)
  y = lax.bitcast_convert_type(i, F32)
  for _ in range(3):
    y = y * (1.5 - 0.5 * x * y * y)
  return y


def _zero_rows(ref, nrows, width):
  z = jnp.zeros((16,), F32)
  nv = width // 16

  @pl.loop(0, nrows)
  def _(r):
    for v in range(nv):
      ref[r, pl.ds(v * 16, 16)] = z


def _edge_sets(edges, ibufa, ibufb, semia, semib, base, nbuf, niter, do_set):
  @pl.loop(0, niter)
  def _(i):
    cb = base + i * (2 * nbuf)

    @pl.when(i == 0)
    def _():
      pltpu.async_copy(edges.at[pl.ds(cb, nbuf)], ibufa, semia)

    pltpu.make_async_copy(edges.at[pl.ds(cb, nbuf)], ibufa, semia).wait()
    hb = pltpu.async_copy(edges.at[pl.ds(cb + nbuf, nbuf)], ibufb, semib)
    do_set(ibufa)
    hb.wait()

    @pl.when(i < niter - 1)
    def _():
      pltpu.async_copy(edges.at[pl.ds(cb + 2 * nbuf, nbuf)], ibufa, semia)

    do_set(ibufb)


@functools.lru_cache(maxsize=None)
def _prep_kernel():
  def body(edges, types, t1l, t1r,
           dinv_out, xwsl, xwsr,
           acc2, zbuf, ones, ibufa, ibufb, dbuf, tbuf, gbuf,
           semia, semib, sems):
    c = lax.axis_index("c")
    s = lax.axis_index("s")

    _zero_rows(zbuf, 256, 16)
    one = jnp.ones((16,), F32)

    @pl.loop(0, 128)
    def _(r):
      ones[r, pl.ds(0, 16)] = one

    @pl.loop(0, 13)
    def _(k):
      pltpu.sync_copy(zbuf, acc2.at[pl.ds((s * 13 + k) * 256, 256)])

    plsc.subcore_barrier()

    def hist_set(ibuf):
      hs = [pltpu.async_copy(ones, acc2.at[ibuf.at[b].at[1]], sems, add=True)
            for b in range(4)]
      for h in hs:
        h.wait()

    _edge_sets(edges, ibufa, ibufb, semia, semib, s * CPT, 4, CPT // 8,
               hist_set)

    plsc.subcore_barrier()

    pltpu.sync_copy(acc2.at[pl.ds(s * NPT, NPT)], dbuf)

    @pl.loop(0, NPT)
    def _(r):
      v = dbuf[r, pl.ds(0, 16)]
      dbuf[r, pl.ds(0, 16)] = _rsqrt16(v + 1.0)

    @pl.when(c == 0)
    def _():
      pltpu.sync_copy(dbuf, dinv_out.at[pl.ds(s * NPT, NPT)])

    def xws_phase(t1, out):
      @pl.loop(0, 25)
      def _(ch):
        row = s * 25 + ch
        pltpu.sync_copy(types.at[row], tbuf)
        pltpu.async_copy(t1.at[tbuf], gbuf, sems).wait()

        @pl.loop(0, 128)
        def _(r):
          v = dbuf[ch * 128 + r, pl.ds(0, 16)]
          gbuf[r, pl.ds(0, 16)] = gbuf[r, pl.ds(0, 16)] * v
          gbuf[r, pl.ds(16, 16)] = gbuf[r, pl.ds(16, 16)] * v

        pltpu.sync_copy(gbuf, out.at[pl.ds(s * NPT + ch * 128, 128)])

    @pl.when(c == 0)
    def _():
      xws_phase(t1l, xwsl)

    @pl.when(c == 1)
    def _():
      xws_phase(t1r, xwsr)

  return pl.kernel(
      body,
      out_type=(
          jax.ShapeDtypeStruct((NPAD, 16), F32),
          jax.ShapeDtypeStruct((NPAD, HH), F32),
          jax.ShapeDtypeStruct((NPAD, HH), F32),
      ),
      mesh=_mesh(),
      scratch_types=[
          pltpu.VMEM_SHARED((ACC2R, 16), F32),
          pltpu.VMEM((256, 16), F32),
          pltpu.VMEM((128, 16), F32),
          pltpu.VMEM((4, 2, 128), I32),
          pltpu.VMEM((4, 2, 128), I32),
          pltpu.VMEM((NPT, 16), F32),
          pltpu.VMEM((128,), I32),
          pltpu.VMEM((128, HH), F32),
          pltpu.SemaphoreType.DMA,
          pltpu.SemaphoreType.DMA,
          pltpu.SemaphoreType.DMA,
      ],
      compiler_params=_sc_params(),
      name="cgnl_prep",
  )


@functools.lru_cache(maxsize=None)
def _conv_kernel(relu: bool, pool: bool):
  nbuf = 2 if pool else 4
  niter = CPT // (2 * nbuf)

  def body(*refs):
    if pool:
      (edges, xwsl, xwsr, dinv, bias, batch, outl, outr, acc,
       ibufa, ibufb) = refs[:11]
      rbufs = refs[11:11 + nbuf]
      (abuf0, abuf1, vbuf0, vbuf1, sbuf, poolt, bbuf0, bbuf1,
       semia, semib, semg, semsc) = refs[11 + nbuf:]
    else:
      (edges, xwsl, xwsr, dinv, bias, outl, outr, acc,
       ibufa, ibufb) = refs[:10]
      rbufs = refs[10:10 + nbuf]
      (abuf0, abuf1, vbuf0, vbuf1, sbuf,
       semia, semib, semg, semsc) = refs[10 + nbuf:]
      bbuf0 = bbuf1 = None

    c = lax.axis_index("c")
    s = lax.axis_index("s")

    pltpu.sync_copy(bias, sbuf)
    if pool:
      _zero_rows(poolt, GP, HH)

    def run_core(xws, out, ci):
      b0 = sbuf[ci, 0, pl.ds(0, 16)]
      b1 = sbuf[ci, 1, pl.ds(0, 16)]

      hi = [pltpu.async_copy(xws.at[pl.ds(s * NPT + k * 128, 128)],
                             acc.at[pl.ds(s * NPT + k * 128, 128)], semg)
            for k in range(25)]
      for h in hi:
        h.wait()

      plsc.subcore_barrier()

      def do_set(ibuf):
        hg = [pltpu.async_copy(xws.at[ibuf.at[b].at[0]], rbufs[b], semg)
              for b in range(nbuf)]
        hs = []
        for b in range(nbuf):
          hg[b].wait()
          hs.append(pltpu.async_copy(rbufs[b], acc.at[ibuf.at[b].at[1]],
                                     semsc, add=True))
        for h in hs:
          h.wait()

      _edge_sets(edges, ibufa, ibufb, semia, semib, s * CPT, nbuf, niter,
                 do_set)

      plsc.subcore_barrier()

      FR = 64
      nfin = NPT // FR

      def _batch_src(ch):
        return batch.at[s * (nfin // 2) + ch // 2].at[pl.ds((ch % 2) * FR, FR)]

      def fire_fin(ch, abuf, vbuf, bbuf):
        nbase = s * NPT + ch * FR
        pltpu.async_copy(acc.at[pl.ds(nbase, FR)], abuf, semsc)
        pltpu.async_copy(dinv.at[pl.ds(nbase, FR)], vbuf, semia)
        if pool:
          pltpu.async_copy(_batch_src(ch), bbuf, semib)

      def wait_fin(ch, abuf, vbuf, bbuf):
        nbase = s * NPT + ch * FR
        pltpu.make_async_copy(acc.at[pl.ds(nbase, FR)], abuf, semsc).wait()
        pltpu.make_async_copy(dinv.at[pl.ds(nbase, FR)], vbuf, semia).wait()
        if pool:
          pltpu.make_async_copy(_batch_src(ch), bbuf, semib).wait()

      def compute_fin(ch, abuf, vbuf, bbuf):
        nbase = s * NPT + ch * FR

        def row_h(r):
          d = vbuf[r, pl.ds(0, 16)]
          v0 = abuf[r, pl.ds(0, 16)] * d + b0
          v1 = abuf[r, pl.ds(16, 16)] * d + b1
          if relu:
            v0 = jnp.maximum(v0, 0.0)
            v1 = jnp.maximum(v1, 0.0)
          return v0, v1

        if pool:
          @pl.loop(0, FR // 16)
          def _(rg):
            gvec = bbuf[pl.ds(rg * 16, 16)]
            for lane in range(16):
              r = rg * 16 + lane
              v0, v1 = row_h(r)
              gidx = gvec[lane]
              poolt[gidx, pl.ds(0, 16)] = poolt[gidx, pl.ds(0, 16)] + v0
              poolt[gidx, pl.ds(16, 16)] = poolt[gidx, pl.ds(16, 16)] + v1
        else:
          @pl.loop(0, FR)
          def _(r):
            v0, v1 = row_h(r)
            abuf[r, pl.ds(0, 16)] = v0
            abuf[r, pl.ds(16, 16)] = v1

          pltpu.sync_copy(abuf, out.at[pl.ds(nbase, FR)])

      @pl.loop(0, nfin // 2)
      def _(i):
        c0 = 2 * i

        @pl.when(i == 0)
        def _():
          fire_fin(c0, abuf0, vbuf0, bbuf0)

        wait_fin(c0, abuf0, vbuf0, bbuf0)
        fire_fin(c0 + 1, abuf1, vbuf1, bbuf1)
        compute_fin(c0, abuf0, vbuf0, bbuf0)
        wait_fin(c0 + 1, abuf1, vbuf1, bbuf1)

        @pl.when(i < nfin // 2 - 1)
        def _():
          fire_fin(c0 + 2, abuf0, vbuf0, bbuf0)

        compute_fin(c0 + 1, abuf1, vbuf1, bbuf1)

      if pool:
        pltpu.sync_copy(poolt, out.at[s])

    @pl.when(c == 0)
    def _():
      run_core(xwsl, outl, 0)

    @pl.when(c == 1)
    def _():
      run_core(xwsr, outr, 1)

  if pool:
    out_type = (jax.ShapeDtypeStruct((16, GP, HH), F32),
                jax.ShapeDtypeStruct((16, GP, HH), F32))
  else:
    out_type = (jax.ShapeDtypeStruct((NPAD, HH), F32),
                jax.ShapeDtypeStruct((NPAD, HH), F32))
  scratch = [
      pltpu.VMEM_SHARED((ACCR, HH), F32),
      pltpu.VMEM((nbuf, 2, 128), I32),
      pltpu.VMEM((nbuf, 2, 128), I32),
  ]
  scratch += [pltpu.VMEM((128, HH), F32) for _ in range(nbuf)]
  scratch += [
      pltpu.VMEM((64, HH), F32),
      pltpu.VMEM((64, HH), F32),
      pltpu.VMEM((64, 16), F32),
      pltpu.VMEM((64, 16), F32),
      pltpu.VMEM((2, 2, 16), F32),
  ]
  if pool:
    scratch += [
        pltpu.VMEM((GP, HH), F32),
        pltpu.VMEM((64,), I32),
        pltpu.VMEM((64,), I32),
    ]
  scratch += [pltpu.SemaphoreType.DMA] * 4

  return pl.kernel(
      body,
      out_type=out_type,
      mesh=_mesh(),
      scratch_types=scratch,
      compiler_params=_sc_params(),
      name=f"cgnl_conv_{int(relu)}{int(pool)}",
  )


def _dg(a, b):
  return lax.dot_general(a, b, (((1,), (1,)), ((), ())),
                         precision=_PREC, preferred_element_type=F32)


@functools.lru_cache(maxsize=None)
def _t1_kernel():
  def body(ea, wa, ec, wc, oa, oc):
    oa[...] = _dg(ea[...], wa[...])
    oc[...] = _dg(ec[...], wc[...])

  return pl.pallas_call(
      body,
      out_shape=(jax.ShapeDtypeStruct((256, D), F32),
                 jax.ShapeDtypeStruct((256, D), F32)),
  )


@functools.lru_cache(maxsize=None)
def _mm_kernel():
  blk = 2048

  def body(hl, hr, w2, dv, ol, orr):
    h = jnp.concatenate([hl[...], hr[...]], axis=1)
    x = _dg(h, w2[...])
    x = x * dv[...][:, 0:1]
    ol[...] = x[:, :HH]
    orr[...] = x[:, HH:]

  nb = NPAD // blk
  return pl.pallas_call(
      body,
      grid=(nb,),
      in_specs=[
          pl.BlockSpec((blk, HH), lambda i: (i, 0)),
          pl.BlockSpec((blk, HH), lambda i: (i, 0)),
          pl.BlockSpec((D, D), lambda i: (0, 0)),
          pl.BlockSpec((blk, 16), lambda i: (i, 0)),
      ],
      out_specs=(pl.BlockSpec((blk, HH), lambda i: (i, 0)),
                 pl.BlockSpec((blk, HH), lambda i: (i, 0))),
      out_shape=(jax.ShapeDtypeStruct((NPAD, HH), F32),
                 jax.ShapeDtypeStruct((NPAD, HH), F32)),
  )


@functools.lru_cache(maxsize=None)
def _head_kernel():
  def body(pal, par, pcl, pcr, ss, semw, semb, f1w, f1b, f2w, f2b,
           lng, lnb, clsw, clsb, out):
    ha = jnp.concatenate([jnp.sum(pal[...], axis=0)[:G],
                          jnp.sum(par[...], axis=0)[:G]], axis=1)
    hc = jnp.concatenate([jnp.sum(pcl[...], axis=0)[:G],
                          jnp.sum(pcr[...], axis=0)[:G]], axis=1)
    f1 = f1w[...]
    z1 = _dg(ha, f1[:, :D]) + _dg(hc, f1[:, D:]) + f1b[...]
    g1 = 1.0 / (1.0 + jnp.exp(-z1))
    hs = g1 * ha + (1.0 - g1) * hc
    hm = jnp.maximum(_dg(ss[...], semw[...]) + semb[...], 0.0)
    f2 = f2w[...]
    z2 = _dg(hs, f2[:, :D]) + _dg(hm, f2[:, D:]) + f2b[...]
    g2 = 1.0 / (1.0 + jnp.exp(-z2))
    h = g2 * hs + (1.0 - g2) * hm
    mu = jnp.mean(h, axis=1, keepdims=True)
    var = jnp.mean((h - mu) ** 2, axis=1, keepdims=True)
    hn = (h - mu) / jnp.sqrt(var + EPS) * lng[...] + lnb[...]
    out[...] = _dg(hn, clsw[...]) + clsb[...]

  return pl.pallas_call(
      body,
      out_shape=jax.ShapeDtypeStruct((G, 2), F32),
  )


def _pack_edges(edge):
  src = jnp.concatenate(
      [edge[0].astype(I32), jnp.full((EPAD - E,), DUMP, I32)])
  dst = jnp.concatenate(
      [edge[1].astype(I32), jnp.full((EPAD - E,), DUMP, I32)])
  return jnp.stack([src.reshape(ECH, 128), dst.reshape(ECH, 128)], axis=1)


def _encoder(edge, types, batch, t1, w2, b1, b2):
  edges = _pack_edges(edge)
  types_r = jnp.pad(types.astype(I32), (0, NPAD - N)).reshape(NPAD // 128, 128)
  batch_r = jnp.pad(batch.astype(I32), (0, NPAD - N),
                    constant_values=G).reshape(NPAD // 128, 128)
  dinv, xw1l, xw1r = _prep_kernel()(edges, types_r, t1[:, :HH], t1[:, HH:])
  h1l, h1r = _conv_kernel(True, False)(
      edges, xw1l, xw1r, dinv, b1.reshape(2, 2, 16))
  xw2l, xw2r = _mm_kernel()(h1l, h1r, w2, dinv)
  pll, plr = _conv_kernel(False, True)(
      edges, xw2l, xw2r, dinv, b2.reshape(2, 2, 16), batch_r)
  return pll, plr


def kernel(ast_type, ast_edge, ast_batch, cfg_type, cfg_edge, cfg_batch,
           struct_sem, ast_emb, ast_W1, ast_b1, ast_W2, ast_b2,
           cfg_emb, cfg_W1, cfg_b1, cfg_W2, cfg_b2,
           sem_W, sem_b, fuse1_W, fuse1_b, fuse2_W, fuse2_b,
           ln_g, ln_b, cls_W, cls_b):
  ea = jnp.pad(ast_emb, ((0, 256 - ast_emb.shape[0]), (0, 0)))
  ec = jnp.pad(cfg_emb, ((0, 256 - cfg_emb.shape[0]), (0, 0)))
  t1a, t1c = _t1_kernel()(ea, ast_W1, ec, cfg_W1)
  pal, par = _encoder(ast_edge, ast_type, ast_batch, t1a, ast_W2,
                      ast_b1, ast_b2)
  pcl, pcr = _encoder(cfg_edge, cfg_type, cfg_batch, t1c, cfg_W2,
                      cfg_b1, cfg_b2)
  return _head_kernel()(
      pal, par, pcl, pcr, struct_sem, sem_W, sem_b.reshape(1, D),
      fuse1_W, fuse1_b.reshape(1, D), fuse2_W, fuse2_b.reshape(1, D),
      ln_g.reshape(1, D), ln_b.reshape(1, D), cls_W, cls_b.reshape(1, 2))
